# Initial kernel scaffold; baseline (speedup 1.0000x reference)
import jax
import jax.numpy as jnp
from jax.experimental import pallas as pl


def kernel(x, edge_index, batch, W1, b1, g1, be1, W2, b2, g2, be2, LW0, Lb0, EW1, Eb1, eg1, ebe1, EW2, Eb2, eg2, ebe2, LW1, Lb1):
    raise NotImplementedError("write your pallas kernel here")


# speedup vs baseline: 1.0833x; 1.0833x over previous
import functools

import jax
import jax.numpy as jnp
from jax import lax
from jax.experimental import pallas as pl
from jax.experimental.pallas import tpu as pltpu
from jax.experimental.pallas import tpu_sc as plsc

NW = 32
EPS = 1e-5


def _node_body(G, H, x_ref, batch_ref, W1_ref, g1_ref, be1_ref, W2_ref,
               g2_ref, be2_ref, LW0_ref, Lb0_ref, EW1_ref,
               P_ref, Q_ref, z0_ref, out0_ref):
    def bn_relu(h, g, b):
        m = jnp.mean(h, axis=0, keepdims=True)
        v = jnp.mean((h - m) * (h - m), axis=0, keepdims=True)
        return jnp.maximum((h - m) * (g / jnp.sqrt(v + EPS)) + b, 0.0)

    x = x_ref[...]
    h = bn_relu(x @ W1_ref[...], g1_ref[...], be1_ref[...])
    h = bn_relu(h @ W2_ref[...], g2_ref[...], be2_ref[...])
    EW1 = EW1_ref[...]
    A = EW1[:H, :]
    B = EW1[H:, :]
    P_ref[...] = h @ (A - B)
    Q_ref[...] = h @ B
    z0 = h @ LW0_ref[...] + Lb0_ref[...]
    z0_ref[...] = z0
    ids = batch_ref[...]
    n = ids.shape[1]
    onehot = (lax.broadcasted_iota(jnp.int32, (G, n), 0) == ids
              ).astype(jnp.float32)
    sums = onehot @ z0
    cnt = jnp.sum(onehot, axis=1, keepdims=True)
    out0_ref[...] = sums / jnp.maximum(cnt, 1.0)


def _run_node(x, batch2d, W1, g1, be1, W2, g2, be2, LW0, Lb0, EW1, G):
    N, D = x.shape
    H = W1.shape[1]
    T = LW0.shape[1]
    return pl.pallas_call(
        functools.partial(_node_body, G, H),
        out_shape=(
            jax.ShapeDtypeStruct((N, H), jnp.float32),
            jax.ShapeDtypeStruct((N, H), jnp.float32),
            jax.ShapeDtypeStruct((N, T), jnp.float32),
            jax.ShapeDtypeStruct((G, T), jnp.float32),
        ),
    )(x, batch2d, W1, g1, be1, W2, g2, be2, LW0, Lb0, EW1)


def _emlp_body(nsteps, E, stats1_ref, eg1_ref, ebe1_ref, eg2_ref, ebe2_ref,
               EW2_ref, U_ref, W_ref, s2t2_ref, acc_ref):
    i = pl.program_id(0)
    sums = jnp.sum(stats1_ref[...], axis=0)
    mean1 = sums[0:1, :] / E
    var1 = sums[1:2, :] / E - mean1 * mean1
    s1 = eg1_ref[...] / jnp.sqrt(var1 + EPS)
    t1 = ebe1_ref[...] - mean1 * s1

    u = U_ref[...]
    v = jnp.maximum(u * s1 + t1, 0.0)
    w = v @ EW2_ref[...]
    W_ref[...] = w

    blk = jnp.concatenate(
        [jnp.sum(w, axis=0, keepdims=True),
         jnp.sum(w * w, axis=0, keepdims=True)], axis=0)

    @pl.when(i == 0)
    def _():
        acc_ref[...] = jnp.zeros_like(acc_ref)

    acc_ref[...] += blk

    @pl.when(i == nsteps - 1)
    def _():
        a = acc_ref[...]
        mean2 = a[0:1, :] / E
        var2 = a[1:2, :] / E - mean2 * mean2
        s2 = eg2_ref[...] / jnp.sqrt(var2 + EPS)
        t2 = ebe2_ref[...] - mean2 * s2
        s2t2_ref[...] = jnp.concatenate([s2, t2], axis=0)


def _run_emlp(stats1, eg1, ebe1, eg2, ebe2, EW2, U):
    E, H = U.shape
    BLK = 4000
    nsteps = E // BLK
    return pl.pallas_call(
        functools.partial(_emlp_body, nsteps, float(E)),
        grid=(nsteps,),
        in_specs=[
            pl.BlockSpec((NW, 2, 32), lambda i: (0, 0, 0)),
            pl.BlockSpec((1, H), lambda i: (0, 0)),
            pl.BlockSpec((1, H), lambda i: (0, 0)),
            pl.BlockSpec((1, H), lambda i: (0, 0)),
            pl.BlockSpec((1, H), lambda i: (0, 0)),
            pl.BlockSpec((H, H), lambda i: (0, 0)),
            pl.BlockSpec((BLK, H), lambda i: (i, 0)),
        ],
        out_specs=[
            pl.BlockSpec((BLK, H), lambda i: (i, 0)),
            pl.BlockSpec((2, H), lambda i: (0, 0)),
        ],
        out_shape=[
            jax.ShapeDtypeStruct((E, H), jnp.float32),
            jax.ShapeDtypeStruct((2, H), jnp.float32),
        ],
        scratch_shapes=[pltpu.VMEM((2, H), jnp.float32)],
    )(stats1, eg1, ebe1, eg2, ebe2, EW2, U)


def _final_body(G, aggp_ref, cntp_ref, z0_ref, batch_ref, LW1_ref, Lb1_ref,
                out0_ref, out_ref, Z_ref):
    agg = aggp_ref[0] + aggp_ref[1]
    cnt = jnp.sum(cntp_ref[...], axis=0)
    h2 = agg / jnp.maximum(cnt, 1.0)[:, None]
    z1 = h2 @ LW1_ref[...] + Lb1_ref[...]
    Z_ref[...] = z0_ref[...] + z1
    ids = batch_ref[...]
    n = ids.shape[1]
    onehot = (lax.broadcasted_iota(jnp.int32, (G, n), 0) == ids
              ).astype(jnp.float32)
    sums = onehot @ z1
    cntg = jnp.sum(onehot, axis=1, keepdims=True)
    out_ref[...] = out0_ref[...] + sums / jnp.maximum(cntg, 1.0)


def _run_final(aggp, cntp, z0, batch2d, LW1, Lb1, out0, G):
    N, T = z0.shape
    return pl.pallas_call(
        functools.partial(_final_body, G),
        out_shape=(
            jax.ShapeDtypeStruct((G, T), jnp.float32),
            jax.ShapeDtypeStruct((N, T), jnp.float32),
        ),
    )(aggp, cntp, z0, batch2d, LW1, Lb1, out0)


def kernel(x, edge_index, batch, W1, b1, g1, be1, W2, b2, g2, be2, LW0, Lb0,
           EW1, Eb1, eg1, ebe1, EW2, Eb2, eg2, ebe2, LW1, Lb1):
    N, D = x.shape
    E = edge_index.shape[1]
    H = W1.shape[1]
    T = LW0.shape[1]
    G = 128

    r = lambda a: a.reshape(1, -1).astype(jnp.float32)
    batch2d = batch.reshape(1, N).astype(jnp.int32)
    src = edge_index[0].astype(jnp.int32)
    dst = edge_index[1].astype(jnp.int32)

    P, Q, z0, out0 = _run_node(x, batch2d, W1, r(g1), r(be1), W2, r(g2),
                               r(be2), LW0, r(Lb0), EW1, G)

    U = P[dst] + Q[src]
    su = jnp.sum(U, axis=0)
    ssq = jnp.sum(U * U, axis=0)
    stats1 = jnp.zeros((NW, 2, 32), jnp.float32).at[0].set(
        jnp.stack([su, ssq]))
    cntp = jnp.zeros((NW, N), jnp.float32).at[0].set(
        jax.ops.segment_sum(jnp.ones((E,), jnp.float32), dst, num_segments=N))

    Wm, s2t2 = _run_emlp(stats1, r(eg1), r(ebe1), r(eg2), r(ebe2), EW2, U)

    M = jnp.maximum(Wm * s2t2[0:1, :] + s2t2[1:2, :], 0.0)
    agg = jax.ops.segment_sum(M, dst, num_segments=N)
    aggp = jnp.zeros((2, N, H), jnp.float32).at[0].set(agg)

    out, Z = _run_final(aggp, cntp, z0, batch2d, LW1, r(Lb1), out0, G)
    return (out, Z)


# speedup vs baseline: 3.7603x; 3.4712x over previous
import functools

import jax
import jax.numpy as jnp
from jax import lax
from jax.experimental import pallas as pl
from jax.experimental.pallas import tpu as pltpu
from jax.experimental.pallas import tpu_sc as plsc

NW = 32
EPS = 1e-5


def _node_body(G, H, x_ref, batch_ref, W1_ref, g1_ref, be1_ref, W2_ref,
               g2_ref, be2_ref, LW0_ref, Lb0_ref, EW1_ref,
               P_ref, Q_ref, z0_ref, out0_ref):
    def bn_relu(h, g, b):
        m = jnp.mean(h, axis=0, keepdims=True)
        v = jnp.mean((h - m) * (h - m), axis=0, keepdims=True)
        return jnp.maximum((h - m) * (g / jnp.sqrt(v + EPS)) + b, 0.0)

    x = x_ref[...]
    h = bn_relu(x @ W1_ref[...], g1_ref[...], be1_ref[...])
    h = bn_relu(h @ W2_ref[...], g2_ref[...], be2_ref[...])
    EW1 = EW1_ref[...]
    A = EW1[:H, :]
    B = EW1[H:, :]
    P_ref[...] = h @ (A - B)
    Q_ref[...] = h @ B
    z0 = h @ LW0_ref[...] + Lb0_ref[...]
    z0_ref[...] = z0
    ids = batch_ref[...]
    n = ids.shape[1]
    onehot = (lax.broadcasted_iota(jnp.int32, (G, n), 0) == ids
              ).astype(jnp.float32)
    sums = onehot @ z0
    cnt = jnp.sum(onehot, axis=1, keepdims=True)
    out0_ref[...] = sums / jnp.maximum(cnt, 1.0)


def _run_node(x, batch2d, W1, g1, be1, W2, g2, be2, LW0, Lb0, EW1, G):
    N, D = x.shape
    H = W1.shape[1]
    T = LW0.shape[1]
    return pl.pallas_call(
        functools.partial(_node_body, G, H),
        out_shape=(
            jax.ShapeDtypeStruct((N, H), jnp.float32),
            jax.ShapeDtypeStruct((N, H), jnp.float32),
            jax.ShapeDtypeStruct((N, T), jnp.float32),
            jax.ShapeDtypeStruct((G, T), jnp.float32),
        ),
    )(x, batch2d, W1, g1, be1, W2, g2, be2, LW0, Lb0, EW1)


GSZ = 128
GPS = 8


def _gather_body(nsup, ng_real, P_ref, Q_ref, dst2_ref, src2_ref,
                 U_ref, stats_ref, idxd, idxs, pd, qs, acc, sem):
    wid = lax.axis_index("c") * 16 + lax.axis_index("s")
    zero16 = jnp.zeros((16,), jnp.float32)
    for r0 in range(2):
        for c0 in range(2):
            acc[r0, pl.ds(c0 * 16, 16)] = zero16

    def super_body(sp, carry):
        gbase = wid * (nsup * GPS) + sp * GPS
        pltpu.sync_copy(dst2_ref.at[pl.ds(gbase, GPS)], idxd)
        pltpu.sync_copy(src2_ref.at[pl.ds(gbase, GPS)], idxs)
        cps = []
        for k in range(GPS):
            row = pl.ds(k * GSZ, GSZ)
            cps.append(pltpu.async_copy(P_ref.at[idxd.at[k]], pd.at[row], sem))
            cps.append(pltpu.async_copy(Q_ref.at[idxs.at[k]], qs.at[row], sem))
        for cp in cps:
            cp.wait()
        for k in range(GPS):
            @pl.when(gbase + k < ng_real)
            def _():
                def row_body(i, c2):
                    for j in range(2):
                        sl = pl.ds(j * 16, 16)
                        u = pd[i, sl] + qs[i, sl]
                        pd[i, sl] = u
                        plsc.addupdate(acc.at[0, sl], u)
                        plsc.addupdate(acc.at[1, sl], u * u)
                    return c2
                lax.fori_loop(k * GSZ, (k + 1) * GSZ, row_body, 0)
        pltpu.sync_copy(pd, U_ref.at[pl.ds(gbase * GSZ, GPS * GSZ)])
        return carry

    lax.fori_loop(0, nsup, super_body, 0)
    pltpu.sync_copy(acc, stats_ref.at[wid])


def _run_gather(P, Q, dst2, src2, ng_real, nsup):
    N, H = P.shape
    epad = dst2.shape[0] * GSZ
    mesh = plsc.VectorSubcoreMesh(core_axis_name="c", subcore_axis_name="s",
                                  num_cores=2, num_subcores=16)
    return pl.kernel(
        functools.partial(_gather_body, nsup, ng_real),
        out_type=(
            jax.ShapeDtypeStruct((epad, H), jnp.float32),
            jax.ShapeDtypeStruct((NW, 2, H), jnp.float32),
        ),
        mesh=mesh,
        compiler_params=pltpu.CompilerParams(use_tc_tiling_on_sc=False),
        scratch_types=[
            pltpu.VMEM((GPS, GSZ), jnp.int32),
            pltpu.VMEM((GPS, GSZ), jnp.int32),
            pltpu.VMEM((GPS * GSZ, H), jnp.float32),
            pltpu.VMEM((GPS * GSZ, H), jnp.float32),
            pltpu.VMEM((2, H), jnp.float32),
            pltpu.SemaphoreType.DMA,
        ],
    )(P, Q, dst2, src2)


def _emlp_body(nsteps, E, stats1_ref, eg1_ref, ebe1_ref, eg2_ref, ebe2_ref,
               EW2_ref, U_ref, W_ref, s2t2_ref, acc_ref):
    i = pl.program_id(0)
    sums = jnp.sum(stats1_ref[...], axis=0)
    mean1 = sums[0:1, :] / E
    var1 = sums[1:2, :] / E - mean1 * mean1
    s1 = eg1_ref[...] / jnp.sqrt(var1 + EPS)
    t1 = ebe1_ref[...] - mean1 * s1

    u = U_ref[...]
    v = jnp.maximum(u * s1 + t1, 0.0)
    w = v @ EW2_ref[...]
    W_ref[...] = w

    blk = jnp.concatenate(
        [jnp.sum(w, axis=0, keepdims=True),
         jnp.sum(w * w, axis=0, keepdims=True)], axis=0)

    @pl.when(i == 0)
    def _():
        acc_ref[...] = jnp.zeros_like(acc_ref)

    acc_ref[...] += blk

    @pl.when(i == nsteps - 1)
    def _():
        a = acc_ref[...]
        mean2 = a[0:1, :] / E
        var2 = a[1:2, :] / E - mean2 * mean2
        s2 = eg2_ref[...] / jnp.sqrt(var2 + EPS)
        t2 = ebe2_ref[...] - mean2 * s2
        s2t2_ref[...] = jnp.concatenate([s2, t2], axis=0)


def _run_emlp(stats1, eg1, ebe1, eg2, ebe2, EW2, U, E):
    epad, H = U.shape
    BLK = 4000
    nsteps = E // BLK
    return pl.pallas_call(
        functools.partial(_emlp_body, nsteps, float(E)),
        grid=(nsteps,),
        in_specs=[
            pl.BlockSpec((NW, 2, 32), lambda i: (0, 0, 0)),
            pl.BlockSpec((1, H), lambda i: (0, 0)),
            pl.BlockSpec((1, H), lambda i: (0, 0)),
            pl.BlockSpec((1, H), lambda i: (0, 0)),
            pl.BlockSpec((1, H), lambda i: (0, 0)),
            pl.BlockSpec((H, H), lambda i: (0, 0)),
            pl.BlockSpec((BLK, H), lambda i: (i, 0)),
        ],
        out_specs=[
            pl.BlockSpec((BLK, H), lambda i: (i, 0)),
            pl.BlockSpec((2, H), lambda i: (0, 0)),
        ],
        out_shape=[
            jax.ShapeDtypeStruct((epad, H), jnp.float32),
            jax.ShapeDtypeStruct((2, H), jnp.float32),
        ],
        scratch_shapes=[pltpu.VMEM((2, H), jnp.float32)],
    )(stats1, eg1, ebe1, eg2, ebe2, EW2, U)


def _scatter_body(nsup, ng_real, n_nodes, W_ref, dst2_ref, s2t2_ref,
                  agg_out, cnt_out, agg_sh, cnt_sh, wv, onesv, idxd,
                  s2t2_v, zb32, zb16, sem):
    cid = lax.axis_index("c")
    sid = lax.axis_index("s")
    rpt = n_nodes // 16

    pltpu.sync_copy(s2t2_ref, s2t2_v)

    zero16 = jnp.zeros((16,), jnp.float32)
    one16 = jnp.ones((16,), jnp.float32)

    def fill_rows(r, c2):
        for j in range(2):
            zb32[r, pl.ds(j * 16, 16)] = zero16
        zb16[r] = zero16
        return c2
    lax.fori_loop(0, 125, fill_rows, 0)

    def fill_ones(r, c2):
        onesv[r] = one16
        return c2
    lax.fori_loop(0, GSZ, fill_ones, 0)

    for q in range(5):
        off = sid * rpt + q * 125
        pltpu.sync_copy(zb32, agg_sh.at[pl.ds(off, 125)])
        pltpu.sync_copy(zb16, cnt_sh.at[pl.ds(off, 125)])
    plsc.subcore_barrier()

    def super_body(sp, carry):
        wid = cid * 16 + sid
        gbase = wid * (nsup * GPS) + sp * GPS
        pltpu.sync_copy(dst2_ref.at[pl.ds(gbase, GPS)], idxd)
        pltpu.sync_copy(W_ref.at[pl.ds(gbase * GSZ, GPS * GSZ)], wv)
        for k in range(GPS):
            @pl.when(gbase + k < ng_real)
            def _():
                def row_body(i, c2):
                    for j in range(2):
                        sl = pl.ds(j * 16, 16)
                        s2 = s2t2_v[0, sl]
                        t2 = s2t2_v[1, sl]
                        wv[i, sl] = jnp.maximum(wv[i, sl] * s2 + t2, 0.0)
                    return c2
                lax.fori_loop(k * GSZ, (k + 1) * GSZ, row_body, 0)
                pltpu.sync_copy(wv.at[pl.ds(k * GSZ, GSZ)],
                                agg_sh.at[idxd.at[k]], add=True)
                pltpu.sync_copy(onesv, cnt_sh.at[idxd.at[k]], add=True)
        return carry

    lax.fori_loop(0, nsup, super_body, 0)
    plsc.subcore_barrier()

    for q in range(5):
        off = sid * rpt + q * 125
        pltpu.sync_copy(agg_sh.at[pl.ds(off, 125)], wv.at[pl.ds(0, 125)])
        pltpu.sync_copy(wv.at[pl.ds(0, 125)], agg_out.at[cid, pl.ds(off, 125)])
        pltpu.sync_copy(cnt_sh.at[pl.ds(off, 125)], zb16)
        pltpu.sync_copy(zb16, cnt_out.at[cid, pl.ds(off, 125)])


def _run_scatter(Wm, dst2, s2t2, n_nodes, ng_real, nsup):
    H = Wm.shape[1]
    mesh = plsc.VectorSubcoreMesh(core_axis_name="c", subcore_axis_name="s",
                                  num_cores=2, num_subcores=16)
    return pl.kernel(
        functools.partial(_scatter_body, nsup, ng_real, n_nodes),
        out_type=(
            jax.ShapeDtypeStruct((2, n_nodes, H), jnp.float32),
            jax.ShapeDtypeStruct((2, n_nodes, 16), jnp.float32),
        ),
        mesh=mesh,
        compiler_params=pltpu.CompilerParams(use_tc_tiling_on_sc=False),
        scratch_types=[
            pltpu.VMEM_SHARED((n_nodes, H), jnp.float32),
            pltpu.VMEM_SHARED((n_nodes, 16), jnp.float32),
            pltpu.VMEM((GPS * GSZ, H), jnp.float32),
            pltpu.VMEM((GSZ, 16), jnp.float32),
            pltpu.VMEM((GPS, GSZ), jnp.int32),
            pltpu.VMEM((2, H), jnp.float32),
            pltpu.VMEM((125, H), jnp.float32),
            pltpu.VMEM((125, 16), jnp.float32),
            pltpu.SemaphoreType.DMA,
        ],
    )(Wm, dst2, s2t2)


def _final_body(G, aggp_ref, cntp_ref, z0_ref, batch_ref, LW1_ref, Lb1_ref,
                out0_ref, out_ref, Z_ref):
    agg = aggp_ref[0] + aggp_ref[1]
    cnt = cntp_ref[0, :, 0:1] + cntp_ref[1, :, 0:1]
    h2 = agg / jnp.maximum(cnt, 1.0)
    z1 = h2 @ LW1_ref[...] + Lb1_ref[...]
    Z_ref[...] = z0_ref[...] + z1
    ids = batch_ref[...]
    n = ids.shape[1]
    onehot = (lax.broadcasted_iota(jnp.int32, (G, n), 0) == ids
              ).astype(jnp.float32)
    sums = onehot @ z1
    cntg = jnp.sum(onehot, axis=1, keepdims=True)
    out_ref[...] = out0_ref[...] + sums / jnp.maximum(cntg, 1.0)


def _run_final(aggp, cntp, z0, batch2d, LW1, Lb1, out0, G):
    N, T = z0.shape
    return pl.pallas_call(
        functools.partial(_final_body, G),
        out_shape=(
            jax.ShapeDtypeStruct((G, T), jnp.float32),
            jax.ShapeDtypeStruct((N, T), jnp.float32),
        ),
    )(aggp, cntp, z0, batch2d, LW1, Lb1, out0)


def kernel(x, edge_index, batch, W1, b1, g1, be1, W2, b2, g2, be2, LW0, Lb0,
           EW1, Eb1, eg1, ebe1, EW2, Eb2, eg2, ebe2, LW1, Lb1):
    N, D = x.shape
    E = edge_index.shape[1]
    H = W1.shape[1]
    T = LW0.shape[1]
    G = 128

    r = lambda a: a.reshape(1, -1).astype(jnp.float32)
    batch2d = batch.reshape(1, N).astype(jnp.int32)
    src = edge_index[0].astype(jnp.int32)
    dst = edge_index[1].astype(jnp.int32)

    P, Q, z0, out0 = _run_node(x, batch2d, W1, r(g1), r(be1), W2, r(g2),
                               r(be2), LW0, r(Lb0), EW1, G)

    ng_real = -(-E // GSZ)
    gpt = -(-ng_real // NW)
    nsup = -(-gpt // GPS)
    ngpad = NW * nsup * GPS
    epad = ngpad * GSZ
    padi = jnp.zeros((epad - E,), jnp.int32)
    dst2 = jnp.concatenate([dst, padi]).reshape(ngpad, GSZ)
    src2 = jnp.concatenate([src, padi]).reshape(ngpad, GSZ)

    U, stats1 = _run_gather(P, Q, dst2, src2, ng_real, nsup)
    Wm, s2t2 = _run_emlp(stats1, r(eg1), r(ebe1), r(eg2), r(ebe2), EW2, U, E)
    aggp, cntp = _run_scatter(Wm, dst2, s2t2, N, ng_real, nsup)
    out, Z = _run_final(aggp, cntp, z0, batch2d, LW1, r(Lb1), out0, G)
    return (out, Z)
